# baseline (speedup 1.0000x reference)
import functools
import math

import jax
import jax.numpy as jnp
from jax import lax
from jax.experimental import pallas as pl
from jax.experimental.pallas import tpu as pltpu
from jax.experimental.pallas import tpu_sc as plsc

F = 26
VOCAB = 100000
D = 32
D_OUT = 128
B = 16384

_NW = 32
_BSUB = 2048
_NB = B // _BSUB


def _sc_gather_body(idxT_hbm, table_hbm, out_hbm, vec_v, idx_v, out_v,
                    isem, osem, vsem):
    nc = 2
    d = lax.axis_index("s") * nc + lax.axis_index("c")

    def idx_drain1():
        pltpu.make_async_copy(idxT_hbm.at[0, pl.ds(0, _BSUB)], idx_v.at[0],
                              isem).wait()

    def out_drain1():
        pltpu.make_async_copy(out_v.at[0], out_hbm.at[0, pl.ds(0, _BSUB)],
                              osem).wait()

    def per_field(f, carry):
        for c0 in range(3):
            pltpu.async_copy(idxT_hbm.at[f, pl.ds(c0 * _BSUB, _BSUB)],
                             idx_v.at[c0], isem)
        with jax.named_scope("vecload"):
            pltpu.sync_copy(table_hbm.at[f, d], vec_v)
        row = f * D + d
        for c in range(_NB):
            t = c % 2
            ti = c % 4
            idx_drain1()
            if c + 3 < _NB:
                pltpu.async_copy(idxT_hbm.at[f, pl.ds((c + 3) * _BSUB, _BSUB)],
                                 idx_v.at[(c + 3) % 4], isem)
            if c >= 2:
                out_drain1()

            def gidx(i, _, t=t, ti=ti):
                base = i * 128
                for u in range(8):
                    sl = pl.ds(base + u * 16, 16)
                    out_v[t, sl] = plsc.load_gather(vec_v, [idx_v[ti, sl]])
                return _

            lax.fori_loop(0, _BSUB // 128, gidx, None)
            pltpu.async_copy(out_v.at[t],
                             out_hbm.at[row, pl.ds(c * _BSUB, _BSUB)], osem)
        with jax.named_scope("taildrain"):
            out_drain1()
            out_drain1()
        return carry

    lax.fori_loop(0, F, per_field, None)


@functools.partial(
    pl.kernel,
    mesh=plsc.VectorSubcoreMesh(core_axis_name="c", subcore_axis_name="s"),
    out_type=jax.ShapeDtypeStruct((F * D, B), jnp.float32),
    scratch_types=[
        pltpu.VMEM((VOCAB,), jnp.float32),
        pltpu.VMEM((4, _BSUB), jnp.int32),
        pltpu.VMEM((2, _BSUB), jnp.float32),
        pltpu.SemaphoreType.DMA,
        pltpu.SemaphoreType.DMA,
        pltpu.SemaphoreType.DMA,
    ],
    compiler_params=pltpu.CompilerParams(use_tc_tiling_on_sc=True,
                                         needs_layout_passes=False),
)
def _sc_gather(idxT_hbm, table_hbm, out_hbm, vec_v, idx_v, out_v, isem, osem,
               vsem):
    _sc_gather_body(idxT_hbm, table_hbm, out_hbm, vec_v, idx_v, out_v,
                    isem, osem, vsem)


_BM = 2048


def _tc_mlp_body(x_ref, w_ref, b_ref, g_ref, be_ref, o_ref):
    y = lax.dot_general(x_ref[...], w_ref[...], (((0,), (0,)), ((), ())),
                        preferred_element_type=jnp.float32) + b_ref[...]
    mu = jnp.mean(y, axis=-1, keepdims=True)
    var = jnp.mean((y - mu) * (y - mu), axis=-1, keepdims=True)
    y = (y - mu) * lax.rsqrt(var + 1e-5)
    y = y * g_ref[...] + be_ref[...]
    o_ref[...] = y * 0.5 * (1.0 + lax.erf(y * (1.0 / math.sqrt(2.0))))


def _tc_mlp(xT, W, b, gamma, beta):
    grid = (B // _BM,)
    return pl.pallas_call(
        _tc_mlp_body,
        grid=grid,
        in_specs=[
            pl.BlockSpec((F * D, _BM), lambda i: (0, i)),
            pl.BlockSpec((F * D, D_OUT), lambda i: (0, 0)),
            pl.BlockSpec((1, D_OUT), lambda i: (0, 0)),
            pl.BlockSpec((1, D_OUT), lambda i: (0, 0)),
            pl.BlockSpec((1, D_OUT), lambda i: (0, 0)),
        ],
        out_specs=pl.BlockSpec((_BM, D_OUT), lambda i: (i, 0)),
        out_shape=jax.ShapeDtypeStruct((B, D_OUT), jnp.float32),
    )(xT, W, b, gamma, beta)


def kernel(batch_factors, emb, W, b, gamma, beta):
    idxT = batch_factors.T
    embT = jnp.swapaxes(emb, 1, 2)
    xT = _sc_gather(idxT, embT)
    out = _tc_mlp(xT, W, b.reshape(1, D_OUT), gamma.reshape(1, D_OUT),
                  beta.reshape(1, D_OUT))
    return (out, jnp.ones((F,), dtype=jnp.float32))


# speedup vs baseline: 1.4010x; 1.4010x over previous
import functools
import math

import jax
import jax.numpy as jnp
from jax import lax
from jax.experimental import pallas as pl
from jax.experimental.pallas import tpu as pltpu
from jax.experimental.pallas import tpu_sc as plsc

F = 26
VOCAB = 100000
D = 32
D_OUT = 128
B = 16384

_NW = 32
_BSUB = 2048
_NB = B // _BSUB


def _sc_gather_body(idxT_hbm, table_hbm, out_hbm, vec_v, idx_v, out_v,
                    isem, osem, vsem):
    nc = 2
    d = lax.axis_index("s") * nc + lax.axis_index("c")

    def idx_drain1():
        pltpu.make_async_copy(idxT_hbm.at[0, pl.ds(0, _BSUB)], idx_v.at[0],
                              isem).wait()

    def out_drain1():
        pltpu.make_async_copy(out_v.at[0], out_hbm.at[0, pl.ds(0, _BSUB)],
                              osem).wait()

    def per_field(f, carry):
        pltpu.async_copy(idxT_hbm.at[f, pl.ds(0, _BSUB)], idx_v.at[0], isem)
        with jax.named_scope("vecload"):
            pltpu.sync_copy(table_hbm.at[f, d], vec_v)
        row = f * D + d
        for c in range(_NB):
            t = c % 2
            idx_drain1()
            if c + 1 < _NB:
                pltpu.async_copy(idxT_hbm.at[f, pl.ds((c + 1) * _BSUB, _BSUB)],
                                 idx_v.at[(c + 1) % 2], isem)
            if c >= 2:
                out_drain1()

            @plsc.parallel_loop(0, _BSUB // 16, unroll=4)
            def gidx(i, t=t):
                sl = pl.ds(i * 16, 16)
                out_v[t, sl] = plsc.load_gather(vec_v, [idx_v[t, sl]])
            pltpu.async_copy(out_v.at[t],
                             out_hbm.at[row, pl.ds(c * _BSUB, _BSUB)], osem)
        with jax.named_scope("taildrain"):
            out_drain1()
            out_drain1()
        return carry

    lax.fori_loop(0, F, per_field, None)


@functools.partial(
    pl.kernel,
    mesh=plsc.VectorSubcoreMesh(core_axis_name="c", subcore_axis_name="s"),
    out_type=jax.ShapeDtypeStruct((F * D, B), jnp.float32),
    scratch_types=[
        pltpu.VMEM((VOCAB,), jnp.float32),
        pltpu.VMEM((2, _BSUB), jnp.int32),
        pltpu.VMEM((2, _BSUB), jnp.float32),
        pltpu.SemaphoreType.DMA,
        pltpu.SemaphoreType.DMA,
        pltpu.SemaphoreType.DMA,
    ],
    compiler_params=pltpu.CompilerParams(use_tc_tiling_on_sc=True,
                                         needs_layout_passes=False),
)
def _sc_gather(idxT_hbm, table_hbm, out_hbm, vec_v, idx_v, out_v, isem, osem,
               vsem):
    _sc_gather_body(idxT_hbm, table_hbm, out_hbm, vec_v, idx_v, out_v,
                    isem, osem, vsem)


_BM = 2048


def _tc_mlp_body(x_ref, w_ref, b_ref, g_ref, be_ref, o_ref):
    y = lax.dot_general(x_ref[...], w_ref[...], (((0,), (0,)), ((), ())),
                        preferred_element_type=jnp.float32) + b_ref[...]
    mu = jnp.mean(y, axis=-1, keepdims=True)
    var = jnp.mean((y - mu) * (y - mu), axis=-1, keepdims=True)
    y = (y - mu) * lax.rsqrt(var + 1e-5)
    y = y * g_ref[...] + be_ref[...]
    o_ref[...] = y * 0.5 * (1.0 + lax.erf(y * (1.0 / math.sqrt(2.0))))


def _tc_mlp(xT, W, b, gamma, beta):
    grid = (B // _BM,)
    return pl.pallas_call(
        _tc_mlp_body,
        grid=grid,
        in_specs=[
            pl.BlockSpec((F * D, _BM), lambda i: (0, i)),
            pl.BlockSpec((F * D, D_OUT), lambda i: (0, 0)),
            pl.BlockSpec((1, D_OUT), lambda i: (0, 0)),
            pl.BlockSpec((1, D_OUT), lambda i: (0, 0)),
            pl.BlockSpec((1, D_OUT), lambda i: (0, 0)),
        ],
        out_specs=pl.BlockSpec((_BM, D_OUT), lambda i: (i, 0)),
        out_shape=jax.ShapeDtypeStruct((B, D_OUT), jnp.float32),
    )(xT, W, b, gamma, beta)


def kernel(batch_factors, emb, W, b, gamma, beta):
    idxT = batch_factors.T
    embT = jnp.swapaxes(emb, 1, 2)
    xT = _sc_gather(idxT, embT)
    out = _tc_mlp(xT, W, b.reshape(1, D_OUT), gamma.reshape(1, D_OUT),
                  beta.reshape(1, D_OUT))
    return (out, jnp.ones((F,), dtype=jnp.float32))


# speedup vs baseline: 1.4053x; 1.0031x over previous
import functools
import math

import jax
import jax.numpy as jnp
from jax import lax
from jax.experimental import pallas as pl
from jax.experimental.pallas import tpu as pltpu
from jax.experimental.pallas import tpu_sc as plsc

F = 26
VOCAB = 100000
D = 32
D_OUT = 128
B = 16384

_NW = 32
_BSUB = 2048
_NB = B // _BSUB


def _sc_gather_body(idxT_hbm, table_hbm, out_hbm, vec_v, idx_v, out_v,
                    isem, osem, vsem):
    nc = 2
    d = lax.axis_index("s") * nc + lax.axis_index("c")

    def idx_drain1():
        pltpu.make_async_copy(idxT_hbm.at[0, pl.ds(0, _BSUB)], idx_v.at[0],
                              isem).wait()

    def out_drain1():
        pltpu.make_async_copy(out_v.at[0], out_hbm.at[0, pl.ds(0, _BSUB)],
                              osem).wait()

    def per_field(f, carry):
        pltpu.async_copy(idxT_hbm.at[f, pl.ds(0, _BSUB)], idx_v.at[0], isem)
        with jax.named_scope("vecload"):
            pltpu.sync_copy(table_hbm.at[f, d], vec_v)
        row = f * D + d
        for c in range(_NB):
            t = c % 2
            idx_drain1()
            if c + 1 < _NB:
                pltpu.async_copy(idxT_hbm.at[f, pl.ds((c + 1) * _BSUB, _BSUB)],
                                 idx_v.at[(c + 1) % 2], isem)
            if c >= 2:
                out_drain1()

            @plsc.parallel_loop(0, _BSUB // 16, unroll=8)
            def gidx(i, t=t):
                sl = pl.ds(i * 16, 16)
                out_v[t, sl] = plsc.load_gather(vec_v, [idx_v[t, sl]])
            pltpu.async_copy(out_v.at[t],
                             out_hbm.at[row, pl.ds(c * _BSUB, _BSUB)], osem)
        with jax.named_scope("taildrain"):
            out_drain1()
            out_drain1()
        return carry

    lax.fori_loop(0, F, per_field, None)


@functools.partial(
    pl.kernel,
    mesh=plsc.VectorSubcoreMesh(core_axis_name="c", subcore_axis_name="s"),
    out_type=jax.ShapeDtypeStruct((F * D, B), jnp.float32),
    scratch_types=[
        pltpu.VMEM((VOCAB,), jnp.float32),
        pltpu.VMEM((2, _BSUB), jnp.int32),
        pltpu.VMEM((2, _BSUB), jnp.float32),
        pltpu.SemaphoreType.DMA,
        pltpu.SemaphoreType.DMA,
        pltpu.SemaphoreType.DMA,
    ],
    compiler_params=pltpu.CompilerParams(use_tc_tiling_on_sc=True,
                                         needs_layout_passes=False),
)
def _sc_gather(idxT_hbm, table_hbm, out_hbm, vec_v, idx_v, out_v, isem, osem,
               vsem):
    _sc_gather_body(idxT_hbm, table_hbm, out_hbm, vec_v, idx_v, out_v,
                    isem, osem, vsem)


_BM = 2048


def _tc_mlp_body(x_ref, w_ref, b_ref, g_ref, be_ref, o_ref):
    y = lax.dot_general(x_ref[...], w_ref[...], (((0,), (0,)), ((), ())),
                        preferred_element_type=jnp.float32) + b_ref[...]
    mu = jnp.mean(y, axis=-1, keepdims=True)
    var = jnp.mean((y - mu) * (y - mu), axis=-1, keepdims=True)
    y = (y - mu) * lax.rsqrt(var + 1e-5)
    y = y * g_ref[...] + be_ref[...]
    o_ref[...] = y * 0.5 * (1.0 + lax.erf(y * (1.0 / math.sqrt(2.0))))


def _tc_mlp(xT, W, b, gamma, beta):
    grid = (B // _BM,)
    return pl.pallas_call(
        _tc_mlp_body,
        grid=grid,
        in_specs=[
            pl.BlockSpec((F * D, _BM), lambda i: (0, i)),
            pl.BlockSpec((F * D, D_OUT), lambda i: (0, 0)),
            pl.BlockSpec((1, D_OUT), lambda i: (0, 0)),
            pl.BlockSpec((1, D_OUT), lambda i: (0, 0)),
            pl.BlockSpec((1, D_OUT), lambda i: (0, 0)),
        ],
        out_specs=pl.BlockSpec((_BM, D_OUT), lambda i: (i, 0)),
        out_shape=jax.ShapeDtypeStruct((B, D_OUT), jnp.float32),
    )(xT, W, b, gamma, beta)


def kernel(batch_factors, emb, W, b, gamma, beta):
    idxT = batch_factors.T
    embT = jnp.swapaxes(emb, 1, 2)
    xT = _sc_gather(idxT, embT)
    out = _tc_mlp(xT, W, b.reshape(1, D_OUT), gamma.reshape(1, D_OUT),
                  beta.reshape(1, D_OUT))
    return (out, jnp.ones((F,), dtype=jnp.float32))


# speedup vs baseline: 1.6863x; 1.1999x over previous
import functools
import math

import jax
import jax.numpy as jnp
from jax import lax
from jax.experimental import pallas as pl
from jax.experimental.pallas import tpu as pltpu
from jax.experimental.pallas import tpu_sc as plsc

F = 26
VOCAB = 100000
D = 32
D_OUT = 128
B = 16384

_NW = 32
_BSUB = 4096
_NB = B // _BSUB


def _sc_gather_body(idxT_hbm, table_hbm, out_hbm, vec_v, idx_v, out_v,
                    isem, osem, vsem):
    nc = 2
    d = lax.axis_index("s") * nc + lax.axis_index("c")

    def idx_drain1():
        pltpu.make_async_copy(idxT_hbm.at[0, pl.ds(0, _BSUB)], idx_v.at[0],
                              isem).wait()

    def out_drain1():
        pltpu.make_async_copy(out_v.at[0], out_hbm.at[0, pl.ds(0, _BSUB)],
                              osem).wait()

    def per_field(f, carry):
        pltpu.async_copy(idxT_hbm.at[f, pl.ds(0, _BSUB)], idx_v.at[0], isem)
        with jax.named_scope("vecload"):
            pltpu.sync_copy(table_hbm.at[f, d], vec_v)
        row = f * D + d
        for c in range(_NB):
            t = c % 2
            idx_drain1()
            if c + 1 < _NB:
                pltpu.async_copy(idxT_hbm.at[f, pl.ds((c + 1) * _BSUB, _BSUB)],
                                 idx_v.at[(c + 1) % 2], isem)
            if c >= 2:
                out_drain1()

            @plsc.parallel_loop(0, _BSUB // 16, unroll=8)
            def gidx(i, t=t):
                sl = pl.ds(i * 16, 16)
                out_v[t, sl] = plsc.load_gather(vec_v, [idx_v[t, sl]])
            pltpu.async_copy(out_v.at[t],
                             out_hbm.at[row, pl.ds(c * _BSUB, _BSUB)], osem)
        with jax.named_scope("taildrain"):
            out_drain1()
            out_drain1()
        return carry

    lax.fori_loop(0, F, per_field, None)


@functools.partial(
    pl.kernel,
    mesh=plsc.VectorSubcoreMesh(core_axis_name="c", subcore_axis_name="s"),
    out_type=jax.ShapeDtypeStruct((F * D, B), jnp.float32),
    scratch_types=[
        pltpu.VMEM((VOCAB,), jnp.float32),
        pltpu.VMEM((2, _BSUB), jnp.int32),
        pltpu.VMEM((2, _BSUB), jnp.float32),
        pltpu.SemaphoreType.DMA,
        pltpu.SemaphoreType.DMA,
        pltpu.SemaphoreType.DMA,
    ],
    compiler_params=pltpu.CompilerParams(use_tc_tiling_on_sc=True,
                                         needs_layout_passes=False),
)
def _sc_gather(idxT_hbm, table_hbm, out_hbm, vec_v, idx_v, out_v, isem, osem,
               vsem):
    _sc_gather_body(idxT_hbm, table_hbm, out_hbm, vec_v, idx_v, out_v,
                    isem, osem, vsem)


_BM = 2048


def _tc_mlp_body(x_ref, w_ref, b_ref, g_ref, be_ref, o_ref):
    y = lax.dot_general(x_ref[...], w_ref[...], (((0,), (0,)), ((), ())),
                        preferred_element_type=jnp.float32) + b_ref[...]
    mu = jnp.mean(y, axis=-1, keepdims=True)
    var = jnp.mean((y - mu) * (y - mu), axis=-1, keepdims=True)
    y = (y - mu) * lax.rsqrt(var + 1e-5)
    y = y * g_ref[...] + be_ref[...]
    o_ref[...] = y * 0.5 * (1.0 + lax.erf(y * (1.0 / math.sqrt(2.0))))


def _tc_mlp(xT, W, b, gamma, beta):
    grid = (B // _BM,)
    return pl.pallas_call(
        _tc_mlp_body,
        grid=grid,
        in_specs=[
            pl.BlockSpec((F * D, _BM), lambda i: (0, i)),
            pl.BlockSpec((F * D, D_OUT), lambda i: (0, 0)),
            pl.BlockSpec((1, D_OUT), lambda i: (0, 0)),
            pl.BlockSpec((1, D_OUT), lambda i: (0, 0)),
            pl.BlockSpec((1, D_OUT), lambda i: (0, 0)),
        ],
        out_specs=pl.BlockSpec((_BM, D_OUT), lambda i: (i, 0)),
        out_shape=jax.ShapeDtypeStruct((B, D_OUT), jnp.float32),
    )(xT, W, b, gamma, beta)


def kernel(batch_factors, emb, W, b, gamma, beta):
    idxT = batch_factors.T
    embT = jnp.swapaxes(emb, 1, 2)
    xT = _sc_gather(idxT, embT)
    out = _tc_mlp(xT, W, b.reshape(1, D_OUT), gamma.reshape(1, D_OUT),
                  beta.reshape(1, D_OUT))
    return (out, jnp.ones((F,), dtype=jnp.float32))


# speedup vs baseline: 1.6909x; 1.0027x over previous
import functools
import math

import jax
import jax.numpy as jnp
from jax import lax
from jax.experimental import pallas as pl
from jax.experimental.pallas import tpu as pltpu
from jax.experimental.pallas import tpu_sc as plsc

F = 26
VOCAB = 100000
D = 32
D_OUT = 128
B = 16384

_NW = 32
_BSUB = 4096
_NB = B // _BSUB


def _sc_gather_body(idxT_hbm, table_hbm, out_hbm, vec_v, idx_v, out_v,
                    isem, osem, vsem):
    nc = 2
    d = lax.axis_index("s") * nc + lax.axis_index("c")

    def idx_drain1():
        pltpu.make_async_copy(idxT_hbm.at[0, pl.ds(0, _BSUB)], idx_v.at[0],
                              isem).wait()

    def out_drain1():
        pltpu.make_async_copy(out_v.at[0], out_hbm.at[0, pl.ds(0, _BSUB)],
                              osem).wait()

    def vec_start(f):
        pltpu.async_copy(table_hbm.at[f, d], vec_v, vsem)

    def vec_wait():
        pltpu.make_async_copy(table_hbm.at[0, 0], vec_v, vsem).wait()

    def per_field(f, carry):
        with jax.named_scope("vecwait"):
            vec_wait()
        row = f * D + d
        for c in range(_NB):
            t = c % 2
            idx_drain1()
            if c + 1 < _NB:
                pltpu.async_copy(idxT_hbm.at[f, pl.ds((c + 1) * _BSUB, _BSUB)],
                                 idx_v.at[(c + 1) % 2], isem)
            if c >= 2:
                out_drain1()

            @plsc.parallel_loop(0, _BSUB // 16, unroll=8)
            def gidx(i, t=t):
                sl = pl.ds(i * 16, 16)
                out_v[t, sl] = plsc.load_gather(vec_v, [idx_v[t, sl]])
            pltpu.async_copy(out_v.at[t],
                             out_hbm.at[row, pl.ds(c * _BSUB, _BSUB)], osem)

        @pl.when(f + 1 < F)
        def _prefetch_next():
            pltpu.async_copy(idxT_hbm.at[f + 1, pl.ds(0, _BSUB)], idx_v.at[0],
                             isem)
            vec_start(f + 1)

        with jax.named_scope("taildrain"):
            out_drain1()
            out_drain1()
        return carry

    pltpu.async_copy(idxT_hbm.at[0, pl.ds(0, _BSUB)], idx_v.at[0], isem)
    vec_start(0)
    lax.fori_loop(0, F, per_field, None)


@functools.partial(
    pl.kernel,
    mesh=plsc.VectorSubcoreMesh(core_axis_name="c", subcore_axis_name="s"),
    out_type=jax.ShapeDtypeStruct((F * D, B), jnp.float32),
    scratch_types=[
        pltpu.VMEM((VOCAB,), jnp.float32),
        pltpu.VMEM((2, _BSUB), jnp.int32),
        pltpu.VMEM((2, _BSUB), jnp.float32),
        pltpu.SemaphoreType.DMA,
        pltpu.SemaphoreType.DMA,
        pltpu.SemaphoreType.DMA,
    ],
    compiler_params=pltpu.CompilerParams(use_tc_tiling_on_sc=True,
                                         needs_layout_passes=False),
)
def _sc_gather(idxT_hbm, table_hbm, out_hbm, vec_v, idx_v, out_v, isem, osem,
               vsem):
    _sc_gather_body(idxT_hbm, table_hbm, out_hbm, vec_v, idx_v, out_v,
                    isem, osem, vsem)


_BM = 2048


def _tc_mlp_body(x_ref, w_ref, b_ref, g_ref, be_ref, o_ref):
    y = lax.dot_general(x_ref[...], w_ref[...], (((0,), (0,)), ((), ())),
                        preferred_element_type=jnp.float32) + b_ref[...]
    mu = jnp.mean(y, axis=-1, keepdims=True)
    var = jnp.mean((y - mu) * (y - mu), axis=-1, keepdims=True)
    y = (y - mu) * lax.rsqrt(var + 1e-5)
    y = y * g_ref[...] + be_ref[...]
    o_ref[...] = y * 0.5 * (1.0 + lax.erf(y * (1.0 / math.sqrt(2.0))))


def _tc_mlp(xT, W, b, gamma, beta):
    grid = (B // _BM,)
    return pl.pallas_call(
        _tc_mlp_body,
        grid=grid,
        in_specs=[
            pl.BlockSpec((F * D, _BM), lambda i: (0, i)),
            pl.BlockSpec((F * D, D_OUT), lambda i: (0, 0)),
            pl.BlockSpec((1, D_OUT), lambda i: (0, 0)),
            pl.BlockSpec((1, D_OUT), lambda i: (0, 0)),
            pl.BlockSpec((1, D_OUT), lambda i: (0, 0)),
        ],
        out_specs=pl.BlockSpec((_BM, D_OUT), lambda i: (i, 0)),
        out_shape=jax.ShapeDtypeStruct((B, D_OUT), jnp.float32),
    )(xT, W, b, gamma, beta)


def kernel(batch_factors, emb, W, b, gamma, beta):
    idxT = batch_factors.T
    embT = jnp.swapaxes(emb, 1, 2)
    xT = _sc_gather(idxT, embT)
    out = _tc_mlp(xT, W, b.reshape(1, D_OUT), gamma.reshape(1, D_OUT),
                  beta.reshape(1, D_OUT))
    return (out, jnp.ones((F,), dtype=jnp.float32))
